# Initial kernel scaffold; baseline (speedup 1.0000x reference)
import jax
import jax.numpy as jnp
from jax.experimental import pallas as pl


def kernel(x, edge_index, W1, b1, W2, b2):
    raise NotImplementedError("write your pallas kernel here")


# speedup vs baseline: 22.0003x; 22.0003x over previous
import functools

import jax
import jax.numpy as jnp
from jax import lax
from jax.experimental import pallas as pl
from jax.experimental.pallas import tpu as pltpu
from jax.experimental.pallas import tpu_sc as plsc

_N = 10000
_E = 320000
_DIN = 128
_DH = 256
_DOUT = 64

_NC = 2
_NS = 16
_NW = _NC * _NS
_EPW = _E // _NW
_CH = 80
_NCH = _EPW // _CH
_NP = 10240
_RPT = _NP // _NS


def _mesh():
    return plsc.VectorSubcoreMesh(core_axis_name="c", subcore_axis_name="s")


def _make_deg():
    @functools.partial(
        pl.kernel,
        out_type=jax.ShapeDtypeStruct((_NW, 1, _NP), jnp.float32),
        mesh=_mesh(),
        compiler_params=pltpu.CompilerParams(needs_layout_passes=False),
        scratch_types=[
            pltpu.VMEM((_EPW,), jnp.int32),
            pltpu.VMEM((_NP,), jnp.float32),
        ],
    )
    def deg_kernel(col_hbm, degp_hbm, col_v, deg_v):
        c = lax.axis_index("c")
        s = lax.axis_index("s")
        wid = s * _NC + c
        pltpu.sync_copy(col_hbm.at[pl.ds(wid * _EPW, _EPW)], col_v)
        zero16 = jnp.zeros((16,), jnp.float32)
        one16 = jnp.ones((16,), jnp.float32)

        def zbody(i, carry):
            deg_v[pl.ds(i * 16, 16)] = zero16
            return carry

        lax.fori_loop(0, _NP // 16, zbody, 0)

        def abody(i, carry):
            idx = col_v[pl.ds(i * 16, 16)]
            plsc.addupdate_scatter(deg_v, [idx], one16)
            return carry

        lax.fori_loop(0, _EPW // 16, abody, 0)
        pltpu.sync_copy(deg_v, degp_hbm.at[wid, 0])

    return deg_kernel


def _make_prop(depth):
    @functools.partial(
        pl.kernel,
        out_type=jax.ShapeDtypeStruct((_NC, _NP, depth), jnp.float32),
        mesh=_mesh(),
        compiler_params=pltpu.CompilerParams(needs_layout_passes=False),
        scratch_types=[
            pltpu.VMEM_SHARED((_NP, depth), jnp.float32),
            pltpu.VMEM((_NCH, _CH), jnp.int32),
            pltpu.VMEM((_NCH, _CH), jnp.int32),
            pltpu.VMEM((_CH, depth), jnp.float32),
            pltpu.SemaphoreType.DMA,
        ],
    )
    def prop_kernel(xs_hbm, row_hbm, col_hbm, z_hbm, up_hbm,
                    u_sh, row_v, col_v, rows_v, gsem):
        c = lax.axis_index("c")
        s = lax.axis_index("s")
        wid = s * _NC + c
        pltpu.sync_copy(row_hbm.at[wid], row_v)
        pltpu.sync_copy(col_hbm.at[wid], col_v)
        pltpu.sync_copy(z_hbm, u_sh.at[pl.ds(s * _RPT, _RPT)])
        plsc.subcore_barrier()

        def body(j, carry):
            pltpu.async_copy(xs_hbm.at[row_v.at[j]], rows_v, gsem).wait()
            pltpu.sync_copy(rows_v, u_sh.at[col_v.at[j]], add=True)
            return carry

        lax.fori_loop(0, _NCH, body, 0)
        plsc.subcore_barrier()
        pltpu.sync_copy(u_sh.at[pl.ds(s * _RPT, _RPT)],
                        up_hbm.at[c].at[pl.ds(s * _RPT, _RPT)])

    return prop_kernel


_R = 1000


_RD = 1024


def _dinv_body(degp_ref, dinv_ref):
    deg = 1.0 + jnp.sum(degp_ref[...], axis=(0, 1))
    dinv_ref[...] = lax.rsqrt(deg)[:, None]


def _make_dinv():
    return pl.pallas_call(
        _dinv_body,
        grid=(_NP // _RD,),
        in_specs=[pl.BlockSpec((_NW, 1, _RD), lambda i: (0, 0, i))],
        out_specs=pl.BlockSpec((_RD, 1), lambda i: (i, 0)),
        out_shape=jax.ShapeDtypeStruct((_NP, 1), jnp.float32),
    )


def _xs_body(x_ref, dinv_ref, xs_ref):
    xs_ref[...] = x_ref[...] * dinv_ref[...]


def _make_xs():
    return pl.pallas_call(
        _xs_body,
        grid=(_N // _R,),
        in_specs=[
            pl.BlockSpec((_R, _DIN), lambda i: (i, 0)),
            pl.BlockSpec((_R, 1), lambda i: (i, 0)),
        ],
        out_specs=pl.BlockSpec((_R, _DIN), lambda i: (i, 0)),
        out_shape=jax.ShapeDtypeStruct((_N, _DIN), jnp.float32),
    )


def _mid_body(up_ref, xs_ref, dinv_ref, w1_ref, b1_ref, w2_ref, ps_ref):
    dinv = dinv_ref[...]
    t = (up_ref[0] + up_ref[1] + xs_ref[...]) * dinv
    h = jnp.dot(t, w1_ref[...], preferred_element_type=jnp.float32) + b1_ref[...]
    h = jnp.maximum(h, 0.0)
    ps = jnp.dot(h * dinv, w2_ref[...], preferred_element_type=jnp.float32)
    ps_ref[...] = jnp.concatenate(
        [ps, jnp.zeros((_R, _DIN - _DOUT), jnp.float32)], axis=1)


def _make_mid():
    return pl.pallas_call(
        _mid_body,
        grid=(_N // _R,),
        in_specs=[
            pl.BlockSpec((_NC, _R, _DIN), lambda i: (0, i, 0)),
            pl.BlockSpec((_R, _DIN), lambda i: (i, 0)),
            pl.BlockSpec((_R, 1), lambda i: (i, 0)),
            pl.BlockSpec((_DIN, _DH), lambda i: (0, 0)),
            pl.BlockSpec((1, _DH), lambda i: (0, 0)),
            pl.BlockSpec((_DH, _DOUT), lambda i: (0, 0)),
        ],
        out_specs=pl.BlockSpec((_R, _DIN), lambda i: (i, 0)),
        out_shape=jax.ShapeDtypeStruct((_N, _DIN), jnp.float32),
    )


def _fin_body(u2_ref, ps_ref, dinv_ref, b2_ref, out_ref):
    u2 = u2_ref[0, :, :_DOUT] + u2_ref[1, :, :_DOUT] + ps_ref[:, :_DOUT]
    out_ref[...] = u2 * dinv_ref[...] + b2_ref[...]


def _make_fin():
    return pl.pallas_call(
        _fin_body,
        grid=(_N // _R,),
        in_specs=[
            pl.BlockSpec((_NC, _R, _DIN), lambda i: (0, i, 0)),
            pl.BlockSpec((_R, _DIN), lambda i: (i, 0)),
            pl.BlockSpec((_R, 1), lambda i: (i, 0)),
            pl.BlockSpec((1, _DOUT), lambda i: (0, 0)),
        ],
        out_specs=pl.BlockSpec((_R, _DOUT), lambda i: (i, 0)),
        out_shape=jax.ShapeDtypeStruct((_N, _DOUT), jnp.float32),
    )


_deg = _make_deg()
_prop128 = _make_prop(_DIN)
_dinv = _make_dinv()
_xs = _make_xs()
_mid = _make_mid()
_fin = _make_fin()


def kernel(x, edge_index, W1, b1, W2, b2):
    row3 = edge_index[0].reshape(_NW, _NCH, _CH)
    col = edge_index[1]
    col3 = col.reshape(_NW, _NCH, _CH)
    z128 = jnp.zeros((_RPT, _DIN), jnp.float32)

    degp = _deg(col)
    dinv = _dinv(degp)
    xs = _xs(x, dinv)
    up = _prop128(xs, row3, col3, z128)
    ps = _mid(up, xs, dinv, W1, b1.reshape(1, -1), W2)
    u2 = _prop128(ps, row3, col3, z128)
    out = _fin(u2, ps, dinv, b2.reshape(1, -1))
    return out


# speedup vs baseline: 26.6127x; 1.2097x over previous
import functools

import jax
import jax.numpy as jnp
from jax import lax
from jax.experimental import pallas as pl
from jax.experimental.pallas import tpu as pltpu
from jax.experimental.pallas import tpu_sc as plsc

_N = 10000
_E = 320000
_DIN = 128
_DH = 256
_DOUT = 64

_NC = 2
_NS = 16
_NW = _NC * _NS
_EPW = _E // _NW
_CH = 128
_NP = 10240
_EPWP = _NP
_NCH = _EPWP // _CH
_RPT = _NP // _NS


def _mesh():
    return plsc.VectorSubcoreMesh(core_axis_name="c", subcore_axis_name="s")


def _make_deg():
    @functools.partial(
        pl.kernel,
        out_type=jax.ShapeDtypeStruct((_NW, 1, _NP), jnp.float32),
        mesh=_mesh(),
        compiler_params=pltpu.CompilerParams(needs_layout_passes=False),
        scratch_types=[
            pltpu.VMEM((_EPW,), jnp.int32),
            pltpu.VMEM((_NP,), jnp.float32),
        ],
    )
    def deg_kernel(col_hbm, degp_hbm, col_v, deg_v):
        c = lax.axis_index("c")
        s = lax.axis_index("s")
        wid = s * _NC + c
        pltpu.sync_copy(col_hbm.at[pl.ds(wid * _EPW, _EPW)], col_v)
        zero16 = jnp.zeros((16,), jnp.float32)
        one16 = jnp.ones((16,), jnp.float32)

        def zbody(i, carry):
            deg_v[pl.ds(i * 16, 16)] = zero16
            return carry

        lax.fori_loop(0, _NP // 16, zbody, 0)

        def abody(i, carry):
            idx = col_v[pl.ds(i * 16, 16)]
            plsc.addupdate_scatter(deg_v, [idx], one16)
            return carry

        lax.fori_loop(0, _EPW // 16, abody, 0)
        pltpu.sync_copy(deg_v, degp_hbm.at[wid, 0])

    return deg_kernel


_NB = 2


def _make_prop(depth):
    @functools.partial(
        pl.kernel,
        out_type=jax.ShapeDtypeStruct((_NC, _NP, depth), jnp.float32),
        mesh=_mesh(),
        compiler_params=pltpu.CompilerParams(needs_layout_passes=False),
        scratch_types=[
            pltpu.VMEM_SHARED((_NP, depth), jnp.float32),
            pltpu.VMEM((_NB, _CH), jnp.int32),
            pltpu.VMEM((_NB, _CH), jnp.int32),
            pltpu.VMEM((_NB, _CH, depth), jnp.float32),
            pltpu.SemaphoreType.DMA((_NB,)),
            pltpu.SemaphoreType.DMA((_NB,)),
            pltpu.SemaphoreType.DMA((_NB,)),
        ],
    )
    def prop_kernel(xs_hbm, row_hbm, col_hbm, z_hbm, up_hbm,
                    u_sh, row_ring, col_ring, rows_v, isem, gsem, ssem):
        c = lax.axis_index("c")
        s = lax.axis_index("s")
        wid = s * _NC + c
        pltpu.sync_copy(z_hbm, u_sh.at[pl.ds(s * _RPT, _RPT)])

        def i_start(j, b):
            pltpu.async_copy(row_hbm.at[wid, j], row_ring.at[b], isem.at[b])
            pltpu.async_copy(col_hbm.at[wid, j], col_ring.at[b], isem.at[b])

        def i_wait(j, b):
            pltpu.make_async_copy(
                row_hbm.at[wid, j], row_ring.at[b], isem.at[b]).wait()
            pltpu.make_async_copy(
                col_hbm.at[wid, j], col_ring.at[b], isem.at[b]).wait()

        def g_desc(b):
            return pltpu.make_async_copy(
                xs_hbm.at[row_ring.at[b]], rows_v.at[b], gsem.at[b])

        def s_desc(b):
            return pltpu.make_async_copy(
                rows_v.at[b], u_sh.at[col_ring.at[b]], ssem.at[b])

        for b in range(_NB):
            i_start(b, b)
        plsc.subcore_barrier()

        def stage(base, prefetch):
            for b in range(_NB):
                i_wait(base + b, b)
                g_desc(b).start()
            for b in range(_NB):
                g_desc(b).wait()
                pltpu.async_copy(rows_v.at[b], u_sh.at[col_ring.at[b]],
                                 ssem.at[b], add=True)
            for b in range(_NB):
                s_desc(b).wait()
                if prefetch:
                    i_start(base + _NB + b, b)

        def body(kk, carry):
            stage(kk * _NB, prefetch=True)
            return carry

        lax.fori_loop(0, _NCH // _NB - 1, body, 0)
        stage(_NCH - _NB, prefetch=False)
        plsc.subcore_barrier()
        pltpu.sync_copy(u_sh.at[pl.ds(s * _RPT, _RPT)],
                        up_hbm.at[c].at[pl.ds(s * _RPT, _RPT)])

    return prop_kernel


_R = 1000


_RD = 1024


def _dinv_body(degp_ref, dinv_ref):
    deg = 1.0 + jnp.sum(degp_ref[...], axis=(0, 1))
    dinv_ref[...] = lax.rsqrt(deg)[:, None]


def _make_dinv():
    return pl.pallas_call(
        _dinv_body,
        grid=(_NP // _RD,),
        in_specs=[pl.BlockSpec((_NW, 1, _RD), lambda i: (0, 0, i))],
        out_specs=pl.BlockSpec((_RD, 1), lambda i: (i, 0)),
        out_shape=jax.ShapeDtypeStruct((_NP, 1), jnp.float32),
    )


def _xs_body(x_ref, dinv_ref, xs_ref):
    xs_ref[...] = x_ref[...] * dinv_ref[...]


def _make_xs():
    return pl.pallas_call(
        _xs_body,
        grid=(_N // _R,),
        in_specs=[
            pl.BlockSpec((_R, _DIN), lambda i: (i, 0)),
            pl.BlockSpec((_R, 1), lambda i: (i, 0)),
        ],
        out_specs=pl.BlockSpec((_R, _DIN), lambda i: (i, 0)),
        out_shape=jax.ShapeDtypeStruct((_N, _DIN), jnp.float32),
    )


def _mid_body(up_ref, xs_ref, dinv_ref, w1_ref, b1_ref, w2_ref, ps_ref):
    dinv = dinv_ref[...]
    t = (up_ref[0] + up_ref[1] + xs_ref[...]) * dinv
    h = jnp.dot(t, w1_ref[...], preferred_element_type=jnp.float32) + b1_ref[...]
    h = jnp.maximum(h, 0.0)
    ps = jnp.dot(h * dinv, w2_ref[...], preferred_element_type=jnp.float32)
    ps_ref[...] = jnp.concatenate(
        [ps, jnp.zeros((_R, _DIN - _DOUT), jnp.float32)], axis=1)


def _make_mid():
    return pl.pallas_call(
        _mid_body,
        grid=(_N // _R,),
        in_specs=[
            pl.BlockSpec((_NC, _R, _DIN), lambda i: (0, i, 0)),
            pl.BlockSpec((_R, _DIN), lambda i: (i, 0)),
            pl.BlockSpec((_R, 1), lambda i: (i, 0)),
            pl.BlockSpec((_DIN, _DH), lambda i: (0, 0)),
            pl.BlockSpec((1, _DH), lambda i: (0, 0)),
            pl.BlockSpec((_DH, _DOUT), lambda i: (0, 0)),
        ],
        out_specs=pl.BlockSpec((_R, _DIN), lambda i: (i, 0)),
        out_shape=jax.ShapeDtypeStruct((_N, _DIN), jnp.float32),
    )


def _fin_body(u2_ref, ps_ref, dinv_ref, b2_ref, out_ref):
    u2 = u2_ref[0, :, :_DOUT] + u2_ref[1, :, :_DOUT] + ps_ref[:, :_DOUT]
    out_ref[...] = u2 * dinv_ref[...] + b2_ref[...]


def _make_fin():
    return pl.pallas_call(
        _fin_body,
        grid=(_N // _R,),
        in_specs=[
            pl.BlockSpec((_NC, _R, _DIN), lambda i: (0, i, 0)),
            pl.BlockSpec((_R, _DIN), lambda i: (i, 0)),
            pl.BlockSpec((_R, 1), lambda i: (i, 0)),
            pl.BlockSpec((1, _DOUT), lambda i: (0, 0)),
        ],
        out_specs=pl.BlockSpec((_R, _DOUT), lambda i: (i, 0)),
        out_shape=jax.ShapeDtypeStruct((_N, _DOUT), jnp.float32),
    )


_deg = _make_deg()
_prop128 = _make_prop(_DIN)
_dinv = _make_dinv()
_xs = _make_xs()
_mid = _make_mid()
_fin = _make_fin()


def kernel(x, edge_index, W1, b1, W2, b2):
    npad = _EPWP - _EPW
    pad_r = jnp.broadcast_to((jnp.arange(npad, dtype=jnp.int32) * 41) % _N,
                             (_NW, npad))
    pad_c = jnp.broadcast_to(_N + jnp.arange(npad, dtype=jnp.int32),
                             (_NW, npad))
    row3 = jnp.concatenate(
        [edge_index[0].reshape(_NW, _EPW), pad_r], axis=1
    ).reshape(_NW, _NCH, _CH)
    col = edge_index[1]
    col3 = jnp.concatenate(
        [col.reshape(_NW, _EPW), pad_c], axis=1).reshape(_NW, _NCH, _CH)
    z128 = jnp.zeros((_RPT, _DIN), jnp.float32)

    degp = _deg(col)
    dinv = _dinv(degp)
    xs = _xs(x, dinv)
    up = _prop128(xs, row3, col3, z128)
    ps = _mid(up, xs, dinv, W1, b1.reshape(1, -1), W2)
    u2 = _prop128(ps, row3, col3, z128)
    out = _fin(u2, ps, dinv, b2.reshape(1, -1))
    return out


# speedup vs baseline: 37.0698x; 1.3929x over previous
import functools

import jax
import jax.numpy as jnp
from jax import lax
from jax.experimental import pallas as pl
from jax.experimental.pallas import tpu as pltpu
from jax.experimental.pallas import tpu_sc as plsc

_N = 10000
_E = 320000
_DIN = 128
_DH = 256
_DOUT = 64

_NC = 2
_NS = 16
_NW = _NC * _NS
_EPW = _E // _NW
_CH = 128
_NP = 10240
_EPWP = _NP
_NCH = _EPWP // _CH
_RPT = _NP // _NS


def _mesh():
    return plsc.VectorSubcoreMesh(core_axis_name="c", subcore_axis_name="s")


def _make_deg():
    @functools.partial(
        pl.kernel,
        out_type=jax.ShapeDtypeStruct((_NW, 1, _NP), jnp.float32),
        mesh=_mesh(),
        compiler_params=pltpu.CompilerParams(needs_layout_passes=False),
        scratch_types=[
            pltpu.VMEM((_EPW,), jnp.int32),
            pltpu.VMEM((_NP,), jnp.float32),
        ],
    )
    def deg_kernel(col_hbm, degp_hbm, col_v, deg_v):
        c = lax.axis_index("c")
        s = lax.axis_index("s")
        wid = s * _NC + c
        pltpu.sync_copy(col_hbm.at[pl.ds(wid * _EPW, _EPW)], col_v)
        zero16 = jnp.zeros((16,), jnp.float32)
        one16 = jnp.ones((16,), jnp.float32)

        def zbody(i, carry):
            deg_v[pl.ds(i * 16, 16)] = zero16
            return carry

        lax.fori_loop(0, _NP // 16, zbody, 0)

        def abody(i, carry):
            idx = col_v[pl.ds(i * 16, 16)]
            plsc.addupdate_scatter(deg_v, [idx], one16)
            return carry

        lax.fori_loop(0, _EPW // 16, abody, 0)
        pltpu.sync_copy(deg_v, degp_hbm.at[wid, 0])

    return deg_kernel


_NB = 2
_NBI = 4


def _make_prop(depth):
    @functools.partial(
        pl.kernel,
        out_type=jax.ShapeDtypeStruct((_NC, _NP, depth), jnp.float32),
        mesh=_mesh(),
        compiler_params=pltpu.CompilerParams(needs_layout_passes=False),
        scratch_types=[
            pltpu.VMEM_SHARED((_NP, depth), jnp.float32),
            pltpu.VMEM((_NBI, _CH), jnp.int32),
            pltpu.VMEM((_NBI, _CH), jnp.int32),
            pltpu.VMEM((_NB, _CH, depth), jnp.float32),
            pltpu.SemaphoreType.DMA((_NBI,)),
            pltpu.SemaphoreType.DMA((_NB,)),
            pltpu.SemaphoreType.DMA((_NBI,)),
        ],
    )
    def prop_kernel(xs_hbm, row_hbm, col_hbm, z_hbm, up_hbm,
                    u_sh, row_ring, col_ring, rows_v, isem, gsem, ssem):
        c = lax.axis_index("c")
        s = lax.axis_index("s")
        wid = s * _NC + c
        pltpu.sync_copy(z_hbm, u_sh.at[pl.ds(s * _RPT, _RPT)])

        def i_start(j, q):
            pltpu.async_copy(row_hbm.at[wid, j], row_ring.at[q], isem.at[q])
            pltpu.async_copy(col_hbm.at[wid, j], col_ring.at[q], isem.at[q])

        def i_wait(j, q):
            pltpu.make_async_copy(
                row_hbm.at[wid, j], row_ring.at[q], isem.at[q]).wait()
            pltpu.make_async_copy(
                col_hbm.at[wid, j], col_ring.at[q], isem.at[q]).wait()

        def g_start(q, b):
            pltpu.async_copy(
                xs_hbm.at[row_ring.at[q]], rows_v.at[b], gsem.at[b])

        def g_wait(q, b):
            pltpu.make_async_copy(
                xs_hbm.at[row_ring.at[q]], rows_v.at[b], gsem.at[b]).wait()

        def s_start(q, b):
            pltpu.async_copy(rows_v.at[b], u_sh.at[col_ring.at[q]],
                             ssem.at[q], add=True)

        def s_wait(q, b):
            pltpu.make_async_copy(
                rows_v.at[b], u_sh.at[col_ring.at[q]], ssem.at[q]).wait()

        for q in range(_NBI):
            i_start(q, q)
        plsc.subcore_barrier()
        i_wait(0, 0)
        g_start(0, 0)
        i_wait(1, 1)
        g_start(1, 1)
        g_wait(0, 0)
        s_start(0, 0)

        def step(j, b, q, prefetch):
            s_wait((q + 2) % _NBI, b)
            if prefetch:
                i_start(j + 2, (q + 2) % _NBI)
            i_wait(j, q)
            g_start(q, b)
            g_wait((q + 3) % _NBI, 1 - b)
            s_start((q + 3) % _NBI, 1 - b)

        def body(kk, carry):
            j0 = 2 + kk * _NBI
            for u in range(_NBI):
                step(j0 + u, u % _NB, (2 + u) % _NBI, True)
            return carry

        lax.fori_loop(0, (_NCH - 4) // _NBI, body, 0)
        step(_NCH - 2, (_NCH - 2) % _NB, (_NCH - 2) % _NBI, False)
        step(_NCH - 1, (_NCH - 1) % _NB, (_NCH - 1) % _NBI, False)
        qf, bf = (_NCH - 1) % _NBI, (_NCH - 1) % _NB
        g_wait(qf, bf)
        s_start(qf, bf)
        s_wait((qf + 3) % _NBI, 1 - bf)
        s_wait(qf, bf)
        plsc.subcore_barrier()
        pltpu.sync_copy(u_sh.at[pl.ds(s * _RPT, _RPT)],
                        up_hbm.at[c].at[pl.ds(s * _RPT, _RPT)])

    return prop_kernel


_R = 1000


_RD = 1024


def _prep_body(degp_ref, x_ref, dinv_ref, xs_ref):
    deg = 1.0 + jnp.sum(degp_ref[...], axis=(0, 1))
    dinv = lax.rsqrt(deg)[:, None]
    dinv_ref[...] = dinv
    xs_ref[...] = x_ref[...] * dinv


def _make_prep():
    return pl.pallas_call(
        _prep_body,
        grid=(_NP // _RD,),
        in_specs=[
            pl.BlockSpec((_NW, 1, _RD), lambda i: (0, 0, i)),
            pl.BlockSpec((_RD, _DIN), lambda i: (i, 0)),
        ],
        out_specs=[
            pl.BlockSpec((_RD, 1), lambda i: (i, 0)),
            pl.BlockSpec((_RD, _DIN), lambda i: (i, 0)),
        ],
        out_shape=[
            jax.ShapeDtypeStruct((_NP, 1), jnp.float32),
            jax.ShapeDtypeStruct((_N, _DIN), jnp.float32),
        ],
    )


def _mid_body(up_ref, xs_ref, dinv_ref, w1_ref, b1_ref, w2_ref, ps_ref):
    dinv = dinv_ref[...]
    t = (up_ref[0] + up_ref[1] + xs_ref[...]) * dinv
    h = jnp.dot(t, w1_ref[...], preferred_element_type=jnp.float32) + b1_ref[...]
    h = jnp.maximum(h, 0.0)
    ps = jnp.dot(h * dinv, w2_ref[...], preferred_element_type=jnp.float32)
    ps_ref[...] = jnp.concatenate(
        [ps, jnp.zeros((_R, _DIN - _DOUT), jnp.float32)], axis=1)


def _make_mid():
    return pl.pallas_call(
        _mid_body,
        grid=(_N // _R,),
        in_specs=[
            pl.BlockSpec((_NC, _R, _DIN), lambda i: (0, i, 0)),
            pl.BlockSpec((_R, _DIN), lambda i: (i, 0)),
            pl.BlockSpec((_R, 1), lambda i: (i, 0)),
            pl.BlockSpec((_DIN, _DH), lambda i: (0, 0)),
            pl.BlockSpec((1, _DH), lambda i: (0, 0)),
            pl.BlockSpec((_DH, _DOUT), lambda i: (0, 0)),
        ],
        out_specs=pl.BlockSpec((_R, _DIN), lambda i: (i, 0)),
        out_shape=jax.ShapeDtypeStruct((_N, _DIN), jnp.float32),
    )


def _fin_body(u2_ref, ps_ref, dinv_ref, b2_ref, out_ref):
    u2 = u2_ref[0, :, :_DOUT] + u2_ref[1, :, :_DOUT] + ps_ref[:, :_DOUT]
    out_ref[...] = u2 * dinv_ref[...] + b2_ref[...]


def _make_fin():
    return pl.pallas_call(
        _fin_body,
        grid=(_N // _R,),
        in_specs=[
            pl.BlockSpec((_NC, _R, _DIN), lambda i: (0, i, 0)),
            pl.BlockSpec((_R, _DIN), lambda i: (i, 0)),
            pl.BlockSpec((_R, 1), lambda i: (i, 0)),
            pl.BlockSpec((1, _DOUT), lambda i: (0, 0)),
        ],
        out_specs=pl.BlockSpec((_R, _DOUT), lambda i: (i, 0)),
        out_shape=jax.ShapeDtypeStruct((_N, _DOUT), jnp.float32),
    )


_deg = _make_deg()
_prop128 = _make_prop(_DIN)
_prep = _make_prep()
_mid = _make_mid()
_fin = _make_fin()


def kernel(x, edge_index, W1, b1, W2, b2):
    npad = _EPWP - _EPW
    pad_r = jnp.broadcast_to((jnp.arange(npad, dtype=jnp.int32) * 41) % _N,
                             (_NW, npad))
    pad_c = jnp.broadcast_to(_N + jnp.arange(npad, dtype=jnp.int32),
                             (_NW, npad))
    row3 = jnp.concatenate(
        [edge_index[0].reshape(_NW, _EPW), pad_r], axis=1
    ).reshape(_NW, _NCH, _CH)
    col = edge_index[1]
    col3 = jnp.concatenate(
        [col.reshape(_NW, _EPW), pad_c], axis=1).reshape(_NW, _NCH, _CH)
    z128 = jnp.zeros((_RPT, _DIN), jnp.float32)

    degp = _deg(col)
    dinv, xs = _prep(degp, x)
    up = _prop128(xs, row3, col3, z128)
    ps = _mid(up, xs, dinv, W1, b1.reshape(1, -1), W2)
    u2 = _prop128(ps, row3, col3, z128)
    out = _fin(u2, ps, dinv, b2.reshape(1, -1))
    return out


# speedup vs baseline: 37.1869x; 1.0032x over previous
import functools

import jax
import jax.numpy as jnp
from jax import lax
from jax.experimental import pallas as pl
from jax.experimental.pallas import tpu as pltpu
from jax.experimental.pallas import tpu_sc as plsc

_N = 10000
_E = 320000
_DIN = 128
_DH = 256
_DOUT = 64

_NC = 2
_NS = 16
_NW = _NC * _NS
_EPW = _E // _NW
_CH = 128
_NCH = 80
_EPWP = _NCH * _CH
_NP = 10240
_RPT = _NP // _NS


def _mesh():
    return plsc.VectorSubcoreMesh(core_axis_name="c", subcore_axis_name="s")


def _make_deg():
    @functools.partial(
        pl.kernel,
        out_type=jax.ShapeDtypeStruct((_NW, 1, _NP), jnp.float32),
        mesh=_mesh(),
        compiler_params=pltpu.CompilerParams(needs_layout_passes=False),
        scratch_types=[
            pltpu.VMEM((_EPW,), jnp.int32),
            pltpu.VMEM((_NP,), jnp.float32),
        ],
    )
    def deg_kernel(col_hbm, degp_hbm, col_v, deg_v):
        c = lax.axis_index("c")
        s = lax.axis_index("s")
        wid = s * _NC + c
        pltpu.sync_copy(col_hbm.at[pl.ds(wid * _EPW, _EPW)], col_v)
        zero16 = jnp.zeros((16,), jnp.float32)
        one16 = jnp.ones((16,), jnp.float32)

        def zbody(i, carry):
            deg_v[pl.ds(i * 16, 16)] = zero16
            return carry

        lax.fori_loop(0, _NP // 16, zbody, 0)

        def abody(i, carry):
            idx = col_v[pl.ds(i * 16, 16)]
            plsc.addupdate_scatter(deg_v, [idx], one16)
            return carry

        lax.fori_loop(0, _EPW // 16, abody, 0)
        pltpu.sync_copy(deg_v, degp_hbm.at[wid, 0])

    return deg_kernel


_NB = 2
_NBI = 4


def _make_prop(depth):
    @functools.partial(
        pl.kernel,
        out_type=jax.ShapeDtypeStruct((_NC, _NP, depth), jnp.float32),
        mesh=_mesh(),
        compiler_params=pltpu.CompilerParams(needs_layout_passes=False),
        scratch_types=[
            pltpu.VMEM_SHARED((_NP, depth), jnp.float32),
            pltpu.VMEM((_NBI, _CH), jnp.int32),
            pltpu.VMEM((_NBI, _CH), jnp.int32),
            pltpu.VMEM((_NB, _CH, depth), jnp.float32),
            pltpu.SemaphoreType.DMA((_NBI,)),
            pltpu.SemaphoreType.DMA((_NB,)),
            pltpu.SemaphoreType.DMA((_NBI,)),
        ],
    )
    def prop_kernel(xs_hbm, row_hbm, col_hbm, z_hbm, up_hbm,
                    u_sh, row_ring, col_ring, rows_v, isem, gsem, ssem):
        c = lax.axis_index("c")
        s = lax.axis_index("s")
        wid = s * _NC + c

        def i_start(j, q):
            pltpu.async_copy(row_hbm.at[wid, j], row_ring.at[q], isem.at[q])
            pltpu.async_copy(col_hbm.at[wid, j], col_ring.at[q], isem.at[q])

        def i_wait(j, q):
            pltpu.make_async_copy(
                row_hbm.at[wid, j], row_ring.at[q], isem.at[q]).wait()
            pltpu.make_async_copy(
                col_hbm.at[wid, j], col_ring.at[q], isem.at[q]).wait()

        def g_start(q, b):
            pltpu.async_copy(
                xs_hbm.at[row_ring.at[q]], rows_v.at[b], gsem.at[b])

        def g_wait(q, b):
            pltpu.make_async_copy(
                xs_hbm.at[row_ring.at[q]], rows_v.at[b], gsem.at[b]).wait()

        def s_start(q, b):
            pltpu.async_copy(rows_v.at[b], u_sh.at[col_ring.at[q]],
                             ssem.at[q], add=True)

        def s_wait(q, b):
            pltpu.make_async_copy(
                rows_v.at[b], u_sh.at[col_ring.at[q]], ssem.at[q]).wait()

        def step(j, prefetch):
            b, q = j % _NB, j % _NBI
            s_wait((q + _NB) % _NBI, b)
            if prefetch:
                i_start(j + _NB, (q + _NB) % _NBI)
            i_wait(j, q)
            g_start(q, b)
            g_wait((q + _NBI - 1) % _NBI, (b + _NB - 1) % _NB)
            s_start((q + _NBI - 1) % _NBI, (b + _NB - 1) % _NB)

        for q in range(_NBI):
            i_start(q, q)
        pltpu.sync_copy(z_hbm, u_sh.at[pl.ds(s * _RPT, _RPT)])
        i_wait(0, 0)
        g_start(0, 0)
        i_wait(1, 1)
        g_start(1, 1)
        plsc.subcore_barrier()
        g_wait(0, 0)
        s_start(0, 0)

        nmain = ((_NCH - 2 - _NB) // _NBI) * _NBI

        def body(kk, carry):
            j0 = 2 + kk * _NBI
            for u in range(_NBI):
                step(j0 + u, True)
            return carry

        lax.fori_loop(0, nmain // _NBI, body, 0)
        for j in range(2 + nmain, _NCH):
            step(j, j + _NB < _NCH)
        qf, bf = (_NCH - 1) % _NBI, (_NCH - 1) % _NB
        g_wait(qf, bf)
        s_start(qf, bf)
        for j in range(_NCH - _NB, _NCH):
            s_wait(j % _NBI, j % _NB)
        plsc.subcore_barrier()
        pltpu.sync_copy(u_sh.at[pl.ds(s * _RPT, _RPT)],
                        up_hbm.at[c].at[pl.ds(s * _RPT, _RPT)])

    return prop_kernel


_R = 1000


_RD = 1024


def _prep_body(degp_ref, x_ref, dinv_ref, xs_ref):
    deg = 1.0 + jnp.sum(degp_ref[...], axis=(0, 1))
    dinv = lax.rsqrt(deg)[:, None]
    dinv_ref[...] = dinv
    xs_ref[...] = x_ref[...] * dinv


def _make_prep():
    return pl.pallas_call(
        _prep_body,
        grid=(_NP // _RD,),
        in_specs=[
            pl.BlockSpec((_NW, 1, _RD), lambda i: (0, 0, i)),
            pl.BlockSpec((_RD, _DIN), lambda i: (i, 0)),
        ],
        out_specs=[
            pl.BlockSpec((_RD, 1), lambda i: (i, 0)),
            pl.BlockSpec((_RD, _DIN), lambda i: (i, 0)),
        ],
        out_shape=[
            jax.ShapeDtypeStruct((_NP, 1), jnp.float32),
            jax.ShapeDtypeStruct((_N, _DIN), jnp.float32),
        ],
    )


def _mid_body(up_ref, xs_ref, dinv_ref, w1_ref, b1_ref, w2_ref, ps_ref):
    dinv = dinv_ref[...]
    t = (up_ref[0] + up_ref[1] + xs_ref[...]) * dinv
    h = jnp.dot(t, w1_ref[...], preferred_element_type=jnp.float32) + b1_ref[...]
    h = jnp.maximum(h, 0.0)
    ps = jnp.dot(h * dinv, w2_ref[...], preferred_element_type=jnp.float32)
    ps_ref[...] = jnp.concatenate(
        [ps, jnp.zeros((_R, _DIN - _DOUT), jnp.float32)], axis=1)


def _make_mid():
    return pl.pallas_call(
        _mid_body,
        grid=(_N // _R,),
        in_specs=[
            pl.BlockSpec((_NC, _R, _DIN), lambda i: (0, i, 0)),
            pl.BlockSpec((_R, _DIN), lambda i: (i, 0)),
            pl.BlockSpec((_R, 1), lambda i: (i, 0)),
            pl.BlockSpec((_DIN, _DH), lambda i: (0, 0)),
            pl.BlockSpec((1, _DH), lambda i: (0, 0)),
            pl.BlockSpec((_DH, _DOUT), lambda i: (0, 0)),
        ],
        out_specs=pl.BlockSpec((_R, _DIN), lambda i: (i, 0)),
        out_shape=jax.ShapeDtypeStruct((_N, _DIN), jnp.float32),
    )


def _fin_body(u2_ref, ps_ref, dinv_ref, b2_ref, out_ref):
    u2 = u2_ref[0, :, :_DOUT] + u2_ref[1, :, :_DOUT] + ps_ref[:, :_DOUT]
    out_ref[...] = u2 * dinv_ref[...] + b2_ref[...]


def _make_fin():
    return pl.pallas_call(
        _fin_body,
        grid=(_N // _R,),
        in_specs=[
            pl.BlockSpec((_NC, _R, _DIN), lambda i: (0, i, 0)),
            pl.BlockSpec((_R, _DIN), lambda i: (i, 0)),
            pl.BlockSpec((_R, 1), lambda i: (i, 0)),
            pl.BlockSpec((1, _DOUT), lambda i: (0, 0)),
        ],
        out_specs=pl.BlockSpec((_R, _DOUT), lambda i: (i, 0)),
        out_shape=jax.ShapeDtypeStruct((_N, _DOUT), jnp.float32),
    )


_deg = _make_deg()
_prop128 = _make_prop(_DIN)
_prep = _make_prep()
_mid = _make_mid()
_fin = _make_fin()


def kernel(x, edge_index, W1, b1, W2, b2):
    npad = _EPWP - _EPW
    pad_r = jnp.broadcast_to((jnp.arange(npad, dtype=jnp.int32) * 41) % _N,
                             (_NW, npad))
    pad_c = jnp.broadcast_to(_N + jnp.arange(npad, dtype=jnp.int32),
                             (_NW, npad))
    row3 = jnp.concatenate(
        [edge_index[0].reshape(_NW, _EPW), pad_r], axis=1
    ).reshape(_NW, _NCH, _CH)
    col = edge_index[1]
    col3 = jnp.concatenate(
        [col.reshape(_NW, _EPW), pad_c], axis=1).reshape(_NW, _NCH, _CH)
    z128 = jnp.zeros((_RPT, _DIN), jnp.float32)

    degp = _deg(col)
    dinv, xs = _prep(degp, x)
    up = _prop128(xs, row3, col3, z128)
    ps = _mid(up, xs, dinv, W1, b1.reshape(1, -1), W2)
    u2 = _prop128(ps, row3, col3, z128)
    out = _fin(u2, ps, dinv, b2.reshape(1, -1))
    return out


# speedup vs baseline: 37.3945x; 1.0056x over previous
import functools

import jax
import jax.numpy as jnp
from jax import lax
from jax.experimental import pallas as pl
from jax.experimental.pallas import tpu as pltpu
from jax.experimental.pallas import tpu_sc as plsc

_N = 10000
_E = 320000
_DIN = 128
_DH = 256
_DOUT = 64

_NC = 2
_NS = 16
_NW = _NC * _NS
_EPW = _E // _NW
_CH = 128
_NCH = 80
_EPWP = _NCH * _CH
_NP = 10240
_RPT = _NP // _NS


def _mesh():
    return plsc.VectorSubcoreMesh(core_axis_name="c", subcore_axis_name="s")


def _make_deg():
    @functools.partial(
        pl.kernel,
        out_type=jax.ShapeDtypeStruct((_NW, 1, _NP), jnp.float32),
        mesh=_mesh(),
        compiler_params=pltpu.CompilerParams(needs_layout_passes=False),
        scratch_types=[
            pltpu.VMEM((_EPW,), jnp.int32),
            pltpu.VMEM((_NP,), jnp.float32),
        ],
    )
    def deg_kernel(col_hbm, degp_hbm, col_v, deg_v):
        c = lax.axis_index("c")
        s = lax.axis_index("s")
        wid = s * _NC + c
        pltpu.sync_copy(col_hbm.at[pl.ds(wid * _EPW, _EPW)], col_v)
        zero16 = jnp.zeros((16,), jnp.float32)
        one16 = jnp.ones((16,), jnp.float32)

        def zbody(i, carry):
            for u in range(8):
                deg_v[pl.ds((i * 8 + u) * 16, 16)] = zero16
            return carry

        lax.fori_loop(0, _NP // 128, zbody, 0)

        def abody(i, carry):
            for u in range(5):
                idx = col_v[pl.ds((i * 5 + u) * 16, 16)]
                plsc.addupdate_scatter(deg_v, [idx], one16)
            return carry

        lax.fori_loop(0, _EPW // 80, abody, 0)
        pltpu.sync_copy(deg_v, degp_hbm.at[wid, 0])

    return deg_kernel


_NB = 2
_NBI = 4


def _make_prop(depth):
    @functools.partial(
        pl.kernel,
        out_type=jax.ShapeDtypeStruct((_NC, _NP, depth), jnp.float32),
        mesh=_mesh(),
        compiler_params=pltpu.CompilerParams(needs_layout_passes=False),
        scratch_types=[
            pltpu.VMEM_SHARED((_NP, depth), jnp.float32),
            pltpu.VMEM((_NBI, _CH), jnp.int32),
            pltpu.VMEM((_NBI, _CH), jnp.int32),
            pltpu.VMEM((_NB, _CH, depth), jnp.float32),
            pltpu.SemaphoreType.DMA((_NBI,)),
            pltpu.SemaphoreType.DMA((_NB,)),
            pltpu.SemaphoreType.DMA((_NBI,)),
        ],
    )
    def prop_kernel(xs_hbm, row_hbm, col_hbm, z_hbm, up_hbm,
                    u_sh, row_ring, col_ring, rows_v, isem, gsem, ssem):
        c = lax.axis_index("c")
        s = lax.axis_index("s")
        wid = s * _NC + c

        def i_start(j, q):
            pltpu.async_copy(row_hbm.at[wid, j], row_ring.at[q], isem.at[q])
            pltpu.async_copy(col_hbm.at[wid, j], col_ring.at[q], isem.at[q])

        def i_wait(j, q):
            pltpu.make_async_copy(
                row_hbm.at[wid, j], row_ring.at[q], isem.at[q]).wait()
            pltpu.make_async_copy(
                col_hbm.at[wid, j], col_ring.at[q], isem.at[q]).wait()

        def g_start(q, b):
            pltpu.async_copy(
                xs_hbm.at[row_ring.at[q]], rows_v.at[b], gsem.at[b])

        def g_wait(q, b):
            pltpu.make_async_copy(
                xs_hbm.at[row_ring.at[q]], rows_v.at[b], gsem.at[b]).wait()

        def s_start(q, b):
            pltpu.async_copy(rows_v.at[b], u_sh.at[col_ring.at[q]],
                             ssem.at[q], add=True)

        def s_wait(q, b):
            pltpu.make_async_copy(
                rows_v.at[b], u_sh.at[col_ring.at[q]], ssem.at[q]).wait()

        def step(j, prefetch):
            b, q = j % _NB, j % _NBI
            s_wait((q + _NB) % _NBI, b)
            if prefetch:
                i_start(j + _NB, (q + _NB) % _NBI)
            i_wait(j, q)
            g_start(q, b)
            g_wait((q + _NBI - 1) % _NBI, (b + _NB - 1) % _NB)
            s_start((q + _NBI - 1) % _NBI, (b + _NB - 1) % _NB)

        for q in range(_NBI):
            i_start(q, q)
        pltpu.sync_copy(z_hbm, u_sh.at[pl.ds(s * _RPT, _RPT)])
        i_wait(0, 0)
        g_start(0, 0)
        i_wait(1, 1)
        g_start(1, 1)
        plsc.subcore_barrier()
        g_wait(0, 0)
        s_start(0, 0)

        nmain = ((_NCH - 2 - _NB) // _NBI) * _NBI

        def body(kk, carry):
            j0 = 2 + kk * _NBI
            for u in range(_NBI):
                step(j0 + u, True)
            return carry

        lax.fori_loop(0, nmain // _NBI, body, 0)
        for j in range(2 + nmain, _NCH):
            step(j, j + _NB < _NCH)
        qf, bf = (_NCH - 1) % _NBI, (_NCH - 1) % _NB
        g_wait(qf, bf)
        s_start(qf, bf)
        for j in range(_NCH - _NB, _NCH):
            s_wait(j % _NBI, j % _NB)
        plsc.subcore_barrier()
        pltpu.sync_copy(u_sh.at[pl.ds(s * _RPT, _RPT)],
                        up_hbm.at[c].at[pl.ds(s * _RPT, _RPT)])

    return prop_kernel


_R = 1000


_RD = 1024


def _prep_body(degp_ref, x_ref, dinv_ref, xs_ref):
    deg = 1.0 + jnp.sum(degp_ref[...], axis=(0, 1))
    dinv = lax.rsqrt(deg)[:, None]
    dinv_ref[...] = dinv
    xs_ref[...] = x_ref[...] * dinv


def _make_prep():
    return pl.pallas_call(
        _prep_body,
        grid=(_NP // _RD,),
        in_specs=[
            pl.BlockSpec((_NW, 1, _RD), lambda i: (0, 0, i)),
            pl.BlockSpec((_RD, _DIN), lambda i: (i, 0)),
        ],
        out_specs=[
            pl.BlockSpec((_RD, 1), lambda i: (i, 0)),
            pl.BlockSpec((_RD, _DIN), lambda i: (i, 0)),
        ],
        out_shape=[
            jax.ShapeDtypeStruct((_NP, 1), jnp.float32),
            jax.ShapeDtypeStruct((_N, _DIN), jnp.float32),
        ],
    )


def _mid_body(up_ref, xs_ref, dinv_ref, w1_ref, b1_ref, w2_ref, ps_ref):
    dinv = dinv_ref[...]
    t = (up_ref[0] + up_ref[1] + xs_ref[...]) * dinv
    h = jnp.dot(t, w1_ref[...], preferred_element_type=jnp.float32) + b1_ref[...]
    h = jnp.maximum(h, 0.0)
    ps = jnp.dot(h * dinv, w2_ref[...], preferred_element_type=jnp.float32)
    ps_ref[...] = jnp.concatenate(
        [ps, jnp.zeros((_R, _DIN - _DOUT), jnp.float32)], axis=1)


def _make_mid():
    return pl.pallas_call(
        _mid_body,
        grid=(_N // _R,),
        in_specs=[
            pl.BlockSpec((_NC, _R, _DIN), lambda i: (0, i, 0)),
            pl.BlockSpec((_R, _DIN), lambda i: (i, 0)),
            pl.BlockSpec((_R, 1), lambda i: (i, 0)),
            pl.BlockSpec((_DIN, _DH), lambda i: (0, 0)),
            pl.BlockSpec((1, _DH), lambda i: (0, 0)),
            pl.BlockSpec((_DH, _DOUT), lambda i: (0, 0)),
        ],
        out_specs=pl.BlockSpec((_R, _DIN), lambda i: (i, 0)),
        out_shape=jax.ShapeDtypeStruct((_N, _DIN), jnp.float32),
    )


def _fin_body(u2_ref, ps_ref, dinv_ref, b2_ref, out_ref):
    u2 = u2_ref[0, :, :_DOUT] + u2_ref[1, :, :_DOUT] + ps_ref[:, :_DOUT]
    out_ref[...] = u2 * dinv_ref[...] + b2_ref[...]


def _make_fin():
    return pl.pallas_call(
        _fin_body,
        grid=(_N // _R,),
        in_specs=[
            pl.BlockSpec((_NC, _R, _DIN), lambda i: (0, i, 0)),
            pl.BlockSpec((_R, _DIN), lambda i: (i, 0)),
            pl.BlockSpec((_R, 1), lambda i: (i, 0)),
            pl.BlockSpec((1, _DOUT), lambda i: (0, 0)),
        ],
        out_specs=pl.BlockSpec((_R, _DOUT), lambda i: (i, 0)),
        out_shape=jax.ShapeDtypeStruct((_N, _DOUT), jnp.float32),
    )


_deg = _make_deg()
_prop128 = _make_prop(_DIN)
_prep = _make_prep()
_mid = _make_mid()
_fin = _make_fin()


def kernel(x, edge_index, W1, b1, W2, b2):
    npad = _EPWP - _EPW
    pad_r = jnp.broadcast_to((jnp.arange(npad, dtype=jnp.int32) * 41) % _N,
                             (_NW, npad))
    pad_c = jnp.broadcast_to(_N + jnp.arange(npad, dtype=jnp.int32),
                             (_NW, npad))
    row3 = jnp.concatenate(
        [edge_index[0].reshape(_NW, _EPW), pad_r], axis=1
    ).reshape(_NW, _NCH, _CH)
    col = edge_index[1]
    col3 = jnp.concatenate(
        [col.reshape(_NW, _EPW), pad_c], axis=1).reshape(_NW, _NCH, _CH)
    z128 = jnp.zeros((_RPT, _DIN), jnp.float32)

    degp = _deg(col)
    dinv, xs = _prep(degp, x)
    up = _prop128(xs, row3, col3, z128)
    ps = _mid(up, xs, dinv, W1, b1.reshape(1, -1), W2)
    u2 = _prop128(ps, row3, col3, z128)
    out = _fin(u2, ps, dinv, b2.reshape(1, -1))
    return out
